# baseline (speedup 1.0000x reference)
import functools

import jax
import jax.numpy as jnp
from jax.experimental import pallas as pl
from jax.experimental.pallas import tpu as pltpu

_LN_EPS = 1e-5


def _ln_t_kernel(sub, w_ref, gm_ref, b_ref, beta_ref, x_ref, o_ref):
    cb = x_ref.shape[1]
    w_mat = w_ref[...]
    gm_row = gm_ref[...]
    b_col = b_ref[...]
    be_col = beta_ref[...]

    def chunk(i, _):
        c0 = i * sub
        xs = x_ref[:, pl.ds(c0, sub)]
        d = jnp.dot(w_mat, xs, preferred_element_type=jnp.float32)
        d = d + b_col
        var = jnp.dot(gm_row, d * d, preferred_element_type=jnp.float32)
        r = jax.lax.rsqrt(var + _LN_EPS)
        o_ref[:, pl.ds(c0, sub)] = jnp.maximum(d * r + be_col, 0.0
                                               ).astype(o_ref.dtype)
        return _

    jax.lax.fori_loop(0, cb // sub, chunk, None)


@functools.partial(jax.jit, static_argnames=("col_block",))
def _encode(x, w, b, gamma, beta, *, col_block=131072, sub_block=8192):
    batch, a_dim = x.shape
    h_dim = w.shape[1]

    w_c = w - jnp.mean(w, axis=1, keepdims=True)
    b_c = b - jnp.mean(b)

    wg = (w_c * gamma[None, :]).T
    bg_col = (b_c * gamma).reshape(h_dim, 1)
    g2 = jnp.maximum(gamma * gamma, jnp.float32(1e-30))
    gm_row = (1.0 / (h_dim * g2)).reshape(1, h_dim)
    be_col = beta.reshape(h_dim, 1)

    xt = x.T

    cost = pl.CostEstimate(
        flops=2 * batch * a_dim * h_dim,
        transcendentals=batch,
        bytes_accessed=4 * (batch * (a_dim + h_dim) + a_dim * h_dim + 3 * h_dim),
    )

    cb = min(col_block, batch)
    cb = max(128, (cb // 128) * 128)
    sub = min(sub_block, cb)
    while cb % sub:
        sub //= 2
    body = functools.partial(_ln_t_kernel, sub)
    yt = pl.pallas_call(
        body,
        out_shape=jax.ShapeDtypeStruct((h_dim, batch), jnp.float32),
        grid=(pl.cdiv(batch, cb),),
        in_specs=[
            pl.BlockSpec((h_dim, a_dim), lambda i: (0, 0)),
            pl.BlockSpec((1, h_dim), lambda i: (0, 0)),
            pl.BlockSpec((h_dim, 1), lambda i: (0, 0)),
            pl.BlockSpec((h_dim, 1), lambda i: (0, 0)),
            pl.BlockSpec((a_dim, cb), lambda i: (0, i)),
        ],
        out_specs=pl.BlockSpec((h_dim, cb), lambda i: (0, i)),
        compiler_params=pltpu.CompilerParams(
            dimension_semantics=("parallel",),
        ),
        cost_estimate=cost,
    )(wg, gm_row, bg_col, be_col, xt)
    return yt.T


def kernel(x, w, b, gamma, beta):
    return _encode(x, w, b, gamma, beta)


# speedup vs baseline: 1.1877x; 1.1877x over previous
import functools

import jax
import jax.numpy as jnp
from jax.experimental import pallas as pl
from jax.experimental.pallas import tpu as pltpu

_LN_EPS = 1e-5


def _ln_t_kernel(sub, w_ref, p_ref, x_ref, o_ref):
    h_dim = w_ref.shape[1]
    cb = x_ref.shape[1]

    w_mat = w_ref[...]
    pvec = p_ref[...]
    b_row = pvec[0:1, :]
    g_row = pvec[1:2, :]
    be_row = pvec[2:3, :]

    wcg = (w_mat - jnp.mean(w_mat, axis=1, keepdims=True)) * g_row
    bg_row = (b_row - jnp.mean(b_row)) * g_row
    g2 = jnp.maximum(g_row * g_row, jnp.float32(1e-30))
    gm_row = 1.0 / (h_dim * g2)
    bg_col = bg_row.reshape(h_dim, 1)
    be_col = be_row.reshape(h_dim, 1)

    def chunk(i, _):
        c0 = i * sub
        xs = x_ref[:, pl.ds(c0, sub)]
        d = jax.lax.dot_general(wcg, xs, (((0,), (0,)), ((), ())),
                                preferred_element_type=jnp.float32)
        d = d + bg_col
        var = jnp.dot(gm_row, d * d, preferred_element_type=jnp.float32)
        r = jax.lax.rsqrt(var + _LN_EPS)
        o_ref[:, pl.ds(c0, sub)] = jnp.maximum(d * r + be_col, 0.0
                                               ).astype(o_ref.dtype)
        return _

    jax.lax.fori_loop(0, cb // sub, chunk, None)


@functools.partial(jax.jit, static_argnames=("col_block", "sub_block"))
def _encode(x, w, b, gamma, beta, *, col_block=65536, sub_block=16384):
    batch, a_dim = x.shape
    h_dim = w.shape[1]

    params = jnp.stack([b, gamma, beta])
    xt = x.T

    cost = pl.CostEstimate(
        flops=2 * batch * a_dim * h_dim,
        transcendentals=batch,
        bytes_accessed=4 * (batch * (a_dim + h_dim) + a_dim * h_dim + 3 * h_dim),
    )

    cb = min(col_block, batch)
    cb = max(128, (cb // 128) * 128)
    sub = min(sub_block, cb)
    while cb % sub:
        sub //= 2
    body = functools.partial(_ln_t_kernel, sub)
    yt = pl.pallas_call(
        body,
        out_shape=jax.ShapeDtypeStruct((h_dim, batch), jnp.float32),
        grid=(pl.cdiv(batch, cb),),
        in_specs=[
            pl.BlockSpec((a_dim, h_dim), lambda i: (0, 0)),
            pl.BlockSpec((3, h_dim), lambda i: (0, 0)),
            pl.BlockSpec((a_dim, cb), lambda i: (0, i)),
        ],
        out_specs=pl.BlockSpec((h_dim, cb), lambda i: (0, i)),
        compiler_params=pltpu.CompilerParams(
            dimension_semantics=("parallel",),
        ),
        cost_estimate=cost,
    )(w, params, xt)
    return yt.T


def kernel(x, w, b, gamma, beta):
    return _encode(x, w, b, gamma, beta)


# speedup vs baseline: 1.2288x; 1.0347x over previous
import functools

import jax
import jax.numpy as jnp
from jax.experimental import pallas as pl
from jax.experimental.pallas import tpu as pltpu

_LN_EPS = 1e-5


def _ln_t_kernel(sub, w_ref, b_ref, g_ref, be_ref, x_ref, o_ref):
    h_dim = w_ref.shape[1]
    cb = x_ref.shape[1]

    w_mat = w_ref[...]
    b_row = b_ref[...].reshape(1, h_dim)
    g_row = g_ref[...].reshape(1, h_dim)
    be_row = be_ref[...].reshape(1, h_dim)

    wcg = (w_mat - jnp.mean(w_mat, axis=1, keepdims=True)) * g_row
    bg_row = (b_row - jnp.mean(b_row)) * g_row
    g2 = jnp.maximum(g_row * g_row, jnp.float32(1e-30))
    gm_row = 1.0 / (h_dim * g2)
    bg_col = bg_row.reshape(h_dim, 1)
    be_col = be_row.reshape(h_dim, 1)

    def chunk(i, _):
        c0 = i * sub
        xs = x_ref[:, pl.ds(c0, sub)]
        d = jax.lax.dot_general(wcg, xs, (((0,), (0,)), ((), ())),
                                preferred_element_type=jnp.float32)
        d = d + bg_col
        var = jnp.dot(gm_row, d * d, preferred_element_type=jnp.float32)
        r = jax.lax.rsqrt(var + _LN_EPS)
        o_ref[:, pl.ds(c0, sub)] = jnp.maximum(d * r + be_col, 0.0
                                               ).astype(o_ref.dtype)
        return _

    jax.lax.fori_loop(0, cb // sub, chunk, None)


@functools.partial(jax.jit, static_argnames=("col_block", "sub_block"))
def _encode(x, w, b, gamma, beta, *, col_block=65536, sub_block=16384):
    batch, a_dim = x.shape
    h_dim = w.shape[1]

    xt = x.T

    cost = pl.CostEstimate(
        flops=2 * batch * a_dim * h_dim,
        transcendentals=batch,
        bytes_accessed=4 * (batch * (a_dim + h_dim) + a_dim * h_dim + 3 * h_dim),
    )

    cb = min(col_block, batch)
    cb = max(128, (cb // 128) * 128)
    sub = min(sub_block, cb)
    while cb % sub:
        sub //= 2
    body = functools.partial(_ln_t_kernel, sub)
    yt = pl.pallas_call(
        body,
        out_shape=jax.ShapeDtypeStruct((h_dim, batch), jnp.float32),
        grid=(pl.cdiv(batch, cb),),
        in_specs=[
            pl.BlockSpec((a_dim, h_dim), lambda i: (0, 0)),
            pl.BlockSpec((h_dim,), lambda i: (0,)),
            pl.BlockSpec((h_dim,), lambda i: (0,)),
            pl.BlockSpec((h_dim,), lambda i: (0,)),
            pl.BlockSpec((a_dim, cb), lambda i: (0, i)),
        ],
        out_specs=pl.BlockSpec((h_dim, cb), lambda i: (0, i)),
        compiler_params=pltpu.CompilerParams(
            dimension_semantics=("parallel",),
        ),
        cost_estimate=cost,
    )(w, b, gamma, beta, xt)
    return yt.T


def kernel(x, w, b, gamma, beta):
    return _encode(x, w, b, gamma, beta)


# speedup vs baseline: 1.2343x; 1.0044x over previous
import functools

import jax
import jax.numpy as jnp
from jax.experimental import pallas as pl
from jax.experimental.pallas import tpu as pltpu

_LN_EPS = 1e-5
_DEPTH = 4


def _ln_t_kernel(sub, n_chunks, w_ref, b_ref, g_ref, be_ref, x_ref, o_ref,
                 obuf, osem):
    h_dim = w_ref.shape[1]
    step = pl.program_id(0)

    w_mat = w_ref[...]
    b_row = b_ref[...].reshape(1, h_dim)
    g_row = g_ref[...].reshape(1, h_dim)
    be_row = be_ref[...].reshape(1, h_dim)

    wcg = (w_mat - jnp.mean(w_mat, axis=1, keepdims=True)) * g_row
    bg_row = (b_row - jnp.mean(b_row)) * g_row
    g2 = jnp.maximum(g_row * g_row, jnp.float32(1e-30))
    gm_row = 1.0 / (h_dim * g2)
    bg_col = bg_row.reshape(h_dim, 1)
    be_col = be_row.reshape(h_dim, 1)

    def out_cp(slot, col0):
        return pltpu.make_async_copy(
            obuf.at[slot], o_ref.at[:, pl.ds(col0, sub)], osem.at[slot])

    def chunk(i, _):
        gc = step * n_chunks + i
        slot = jax.lax.rem(gc, _DEPTH)
        col0 = gc * sub

        @pl.when(gc >= _DEPTH)
        def _():
            out_cp(slot, (gc - _DEPTH) * sub).wait()

        xs = x_ref[:, pl.ds(i * sub, sub)]
        d = jax.lax.dot_general(wcg, xs, (((0,), (0,)), ((), ())),
                                preferred_element_type=jnp.float32)
        d = d + bg_col
        var = jnp.dot(gm_row, d * d, preferred_element_type=jnp.float32)
        r = jax.lax.rsqrt(var + _LN_EPS)
        obuf[slot] = jnp.maximum(d * r + be_col, 0.0)
        out_cp(slot, col0).start()
        return _

    jax.lax.fori_loop(0, n_chunks, chunk, None)

    @pl.when(step == pl.num_programs(0) - 1)
    def _():
        last = pl.num_programs(0) * n_chunks
        for k in range(_DEPTH):
            gc = last - _DEPTH + k
            out_cp(jax.lax.rem(gc, _DEPTH), gc * sub).wait()


@functools.partial(jax.jit, static_argnames=("col_block", "sub_block"))
def _encode(x, w, b, gamma, beta, *, col_block=65536, sub_block=8192):
    batch, a_dim = x.shape
    h_dim = w.shape[1]

    xt = x.T

    cost = pl.CostEstimate(
        flops=2 * batch * a_dim * h_dim,
        transcendentals=batch,
        bytes_accessed=4 * (batch * (a_dim + h_dim) + a_dim * h_dim + 3 * h_dim),
    )

    cb = min(col_block, batch)
    cb = max(128, (cb // 128) * 128)
    while batch % cb:
        cb //= 2
    sub = min(sub_block, cb)
    while cb % sub:
        sub //= 2
    n_chunks = cb // sub
    body = functools.partial(_ln_t_kernel, sub, n_chunks)
    yt = pl.pallas_call(
        body,
        out_shape=jax.ShapeDtypeStruct((h_dim, batch), jnp.float32),
        grid=(batch // cb,),
        in_specs=[
            pl.BlockSpec((a_dim, h_dim), lambda i: (0, 0)),
            pl.BlockSpec((h_dim,), lambda i: (0,)),
            pl.BlockSpec((h_dim,), lambda i: (0,)),
            pl.BlockSpec((h_dim,), lambda i: (0,)),
            pl.BlockSpec((a_dim, cb), lambda i: (0, i)),
        ],
        out_specs=pl.BlockSpec(memory_space=pltpu.MemorySpace.HBM),
        scratch_shapes=[
            pltpu.VMEM((_DEPTH, h_dim, sub), jnp.float32),
            pltpu.SemaphoreType.DMA((_DEPTH,)),
        ],
        compiler_params=pltpu.CompilerParams(
            dimension_semantics=("arbitrary",),
        ),
        cost_estimate=cost,
    )(w, b, gamma, beta, xt)
    return yt.T


def kernel(x, w, b, gamma, beta):
    return _encode(x, w, b, gamma, beta)


# speedup vs baseline: 1.2548x; 1.0166x over previous
import functools

import jax
import jax.numpy as jnp
from jax.experimental import pallas as pl
from jax.experimental.pallas import tpu as pltpu

_LN_EPS = 1e-5
_DEPTH = 4


def _ln_t_kernel(sub, n_chunks, w_ref, b_ref, g_ref, be_ref, x_ref, o_ref,
                 ibuf, obuf, isem, osem):
    h_dim = w_ref.shape[1]

    w_mat = w_ref[...]
    b_row = b_ref[...].reshape(1, h_dim)
    g_row = g_ref[...].reshape(1, h_dim)
    be_row = be_ref[...].reshape(1, h_dim)

    wcg = (w_mat - jnp.mean(w_mat, axis=1, keepdims=True)) * g_row
    bg_row = (b_row - jnp.mean(b_row)) * g_row
    g2 = jnp.maximum(g_row * g_row, jnp.float32(1e-30))
    gm_row = 1.0 / (h_dim * g2)
    bg_col = bg_row.reshape(h_dim, 1)
    be_col = be_row.reshape(h_dim, 1)

    def in_cp(gc):
        slot = jax.lax.rem(gc, _DEPTH)
        return pltpu.make_async_copy(
            x_ref.at[:, pl.ds(gc * sub, sub)], ibuf.at[slot], isem.at[slot])

    def out_cp(gc):
        slot = jax.lax.rem(gc, _DEPTH)
        return pltpu.make_async_copy(
            obuf.at[slot], o_ref.at[:, pl.ds(gc * sub, sub)], osem.at[slot])

    for k in range(_DEPTH):
        in_cp(k).start()

    def chunk(gc, _):
        slot = jax.lax.rem(gc, _DEPTH)
        in_cp(gc).wait()

        @pl.when(gc >= _DEPTH)
        def _():
            out_cp(gc - _DEPTH).wait()

        xs = ibuf[slot]
        d = jax.lax.dot_general(wcg, xs, (((0,), (0,)), ((), ())),
                                preferred_element_type=jnp.float32)
        d = d + bg_col
        var = jnp.dot(gm_row, d * d, preferred_element_type=jnp.float32)
        r = jax.lax.rsqrt(var + _LN_EPS)
        obuf[slot] = jnp.maximum(d * r + be_col, 0.0)
        out_cp(gc).start()

        @pl.when(gc + _DEPTH < n_chunks)
        def _():
            in_cp(gc + _DEPTH).start()
        return _

    jax.lax.fori_loop(0, n_chunks, chunk, None)

    for k in range(_DEPTH):
        out_cp(n_chunks - _DEPTH + k).wait()


@functools.partial(jax.jit, static_argnames=("sub_block",))
def _encode(x, w, b, gamma, beta, *, sub_block=8192):
    batch, a_dim = x.shape
    h_dim = w.shape[1]

    xt = x.T

    cost = pl.CostEstimate(
        flops=2 * batch * a_dim * h_dim,
        transcendentals=batch,
        bytes_accessed=4 * (batch * (a_dim + h_dim) + a_dim * h_dim + 3 * h_dim),
    )

    sub = min(sub_block, batch)
    while batch % sub:
        sub //= 2
    n_chunks = batch // sub
    body = functools.partial(_ln_t_kernel, sub, n_chunks)
    yt = pl.pallas_call(
        body,
        out_shape=jax.ShapeDtypeStruct((h_dim, batch), jnp.float32),
        in_specs=[
            pl.BlockSpec((a_dim, h_dim), lambda: (0, 0)),
            pl.BlockSpec((h_dim,), lambda: (0,)),
            pl.BlockSpec((h_dim,), lambda: (0,)),
            pl.BlockSpec((h_dim,), lambda: (0,)),
            pl.BlockSpec(memory_space=pltpu.MemorySpace.HBM),
        ],
        out_specs=pl.BlockSpec(memory_space=pltpu.MemorySpace.HBM),
        scratch_shapes=[
            pltpu.VMEM((_DEPTH, a_dim, sub), jnp.float32),
            pltpu.VMEM((_DEPTH, h_dim, sub), jnp.float32),
            pltpu.SemaphoreType.DMA((_DEPTH,)),
            pltpu.SemaphoreType.DMA((_DEPTH,)),
        ],
        cost_estimate=cost,
    )(w, b, gamma, beta, xt)
    return yt.T


def kernel(x, w, b, gamma, beta):
    return _encode(x, w, b, gamma, beta)


# speedup vs baseline: 1.2932x; 1.0307x over previous
import functools

import jax
import jax.numpy as jnp
from jax.experimental import pallas as pl
from jax.experimental.pallas import tpu as pltpu

_LN_EPS = 1e-5
_DEPTH = 8


def _ln_t_kernel(sub, n_chunks, w_ref, b_ref, g_ref, be_ref, x_ref, o_ref,
                 ibuf, obuf, isem, osem):
    h_dim = w_ref.shape[1]

    w_mat = w_ref[...]
    b_row = b_ref[...].reshape(1, h_dim)
    g_row = g_ref[...].reshape(1, h_dim)
    be_row = be_ref[...].reshape(1, h_dim)

    wcg = (w_mat - jnp.mean(w_mat, axis=1, keepdims=True)) * g_row
    bg_row = (b_row - jnp.mean(b_row)) * g_row
    g2 = jnp.maximum(g_row * g_row, jnp.float32(1e-30))
    gm_row = 1.0 / (h_dim * g2)
    bg_col = bg_row.reshape(h_dim, 1)
    be_col = be_row.reshape(h_dim, 1)

    def in_cp(gc):
        slot = jax.lax.rem(gc, _DEPTH)
        return pltpu.make_async_copy(
            x_ref.at[:, pl.ds(gc * sub, sub)], ibuf.at[slot], isem.at[slot])

    def out_cp(gc):
        slot = jax.lax.rem(gc, _DEPTH)
        return pltpu.make_async_copy(
            obuf.at[slot], o_ref.at[:, pl.ds(gc * sub, sub)], osem.at[slot])

    for k in range(_DEPTH):
        in_cp(k).start()

    def chunk(gc, _):
        slot = jax.lax.rem(gc, _DEPTH)
        in_cp(gc).wait()

        @pl.when(gc >= _DEPTH)
        def _():
            out_cp(gc - _DEPTH).wait()

        xs = ibuf[slot]
        d = jax.lax.dot_general(wcg, xs, (((0,), (0,)), ((), ())),
                                preferred_element_type=jnp.float32)
        d = d + bg_col
        var = jnp.dot(gm_row, d * d, preferred_element_type=jnp.float32)
        r = jax.lax.rsqrt(var + _LN_EPS)
        obuf[slot] = jnp.maximum(d * r + be_col, 0.0)
        out_cp(gc).start()

        @pl.when(gc + _DEPTH < n_chunks)
        def _():
            in_cp(gc + _DEPTH).start()
        return _

    jax.lax.fori_loop(0, n_chunks, chunk, None)

    for k in range(_DEPTH):
        out_cp(n_chunks - _DEPTH + k).wait()


@functools.partial(jax.jit, static_argnames=("sub_block",))
def _encode(x, w, b, gamma, beta, *, sub_block=8192):
    batch, a_dim = x.shape
    h_dim = w.shape[1]

    xt = x.T

    cost = pl.CostEstimate(
        flops=2 * batch * a_dim * h_dim,
        transcendentals=batch,
        bytes_accessed=4 * (batch * (a_dim + h_dim) + a_dim * h_dim + 3 * h_dim),
    )

    sub = min(sub_block, batch)
    while batch % sub:
        sub //= 2
    n_chunks = batch // sub
    body = functools.partial(_ln_t_kernel, sub, n_chunks)
    yt = pl.pallas_call(
        body,
        out_shape=jax.ShapeDtypeStruct((h_dim, batch), jnp.float32),
        in_specs=[
            pl.BlockSpec((a_dim, h_dim), lambda: (0, 0)),
            pl.BlockSpec((h_dim,), lambda: (0,)),
            pl.BlockSpec((h_dim,), lambda: (0,)),
            pl.BlockSpec((h_dim,), lambda: (0,)),
            pl.BlockSpec(memory_space=pltpu.MemorySpace.HBM),
        ],
        out_specs=pl.BlockSpec(memory_space=pltpu.MemorySpace.HBM),
        scratch_shapes=[
            pltpu.VMEM((_DEPTH, a_dim, sub), jnp.float32),
            pltpu.VMEM((_DEPTH, h_dim, sub), jnp.float32),
            pltpu.SemaphoreType.DMA((_DEPTH,)),
            pltpu.SemaphoreType.DMA((_DEPTH,)),
        ],
        cost_estimate=cost,
    )(w, b, gamma, beta, xt)
    return yt.T


def kernel(x, w, b, gamma, beta):
    return _encode(x, w, b, gamma, beta)


# speedup vs baseline: 1.4667x; 1.1341x over previous
import functools

import jax
import jax.numpy as jnp
from jax.experimental import pallas as pl
from jax.experimental.pallas import tpu as pltpu

_LN_EPS = 1e-5
_DEPTH = 8


def _ln_t_kernel(sub, n_chunks, w_ref, b_ref, g_ref, be_ref, x_ref, o_ref,
                 ibuf, obuf, isem, osem):
    h_dim = w_ref.shape[1]

    w_mat = w_ref[...]
    b_row = b_ref[...].reshape(1, h_dim)
    g_row = g_ref[...].reshape(1, h_dim)
    be_row = be_ref[...].reshape(1, h_dim)

    wcg = (w_mat - jnp.mean(w_mat, axis=1, keepdims=True)) * g_row
    bg_row = (b_row - jnp.mean(b_row)) * g_row
    g2 = jnp.maximum(g_row * g_row, jnp.float32(1e-30))
    gm_row = 1.0 / (h_dim * g2)
    bg_col = bg_row.reshape(h_dim, 1)
    be_col = be_row.reshape(h_dim, 1)

    def in_cp(gc):
        slot = jax.lax.rem(gc, _DEPTH)
        return pltpu.make_async_copy(
            x_ref.at[:, pl.ds(gc * sub, sub)], ibuf.at[slot], isem.at[slot])

    def out_cp(gc):
        slot = jax.lax.rem(gc, _DEPTH)
        return pltpu.make_async_copy(
            obuf.at[slot], o_ref.at[:, pl.ds(gc * sub, sub)], osem.at[slot])

    for k in range(_DEPTH):
        in_cp(k).start()

    def chunk(gc, _):
        slot = jax.lax.rem(gc, _DEPTH)
        in_cp(gc).wait()

        @pl.when(gc >= _DEPTH)
        def _():
            out_cp(gc - _DEPTH).wait()

        xs = ibuf[slot]
        d = jax.lax.dot_general(wcg, xs, (((0,), (0,)), ((), ())),
                                preferred_element_type=jnp.float32)
        d = d + bg_col
        var = jnp.dot(gm_row, d * d, preferred_element_type=jnp.float32)
        r = jax.lax.rsqrt(var + _LN_EPS)
        obuf[slot] = jnp.maximum(d * r + be_col, 0.0)
        out_cp(gc).start()

        @pl.when(gc + _DEPTH < n_chunks)
        def _():
            in_cp(gc + _DEPTH).start()
        return _

    jax.lax.fori_loop(0, n_chunks, chunk, None)

    for k in range(_DEPTH):
        out_cp(n_chunks - _DEPTH + k).wait()


@functools.partial(jax.jit, static_argnames=("sub_block",))
def _encode(x, w, b, gamma, beta, *, sub_block=16384):
    batch, a_dim = x.shape
    h_dim = w.shape[1]

    xt = x.T

    cost = pl.CostEstimate(
        flops=2 * batch * a_dim * h_dim,
        transcendentals=batch,
        bytes_accessed=4 * (batch * (a_dim + h_dim) + a_dim * h_dim + 3 * h_dim),
    )

    sub = min(sub_block, batch)
    while batch % sub:
        sub //= 2
    n_chunks = batch // sub
    body = functools.partial(_ln_t_kernel, sub, n_chunks)
    yt = pl.pallas_call(
        body,
        out_shape=jax.ShapeDtypeStruct((h_dim, batch), jnp.float32),
        in_specs=[
            pl.BlockSpec((a_dim, h_dim), lambda: (0, 0)),
            pl.BlockSpec((h_dim,), lambda: (0,)),
            pl.BlockSpec((h_dim,), lambda: (0,)),
            pl.BlockSpec((h_dim,), lambda: (0,)),
            pl.BlockSpec(memory_space=pltpu.MemorySpace.HBM),
        ],
        out_specs=pl.BlockSpec(memory_space=pltpu.MemorySpace.HBM),
        scratch_shapes=[
            pltpu.VMEM((_DEPTH, a_dim, sub), jnp.float32),
            pltpu.VMEM((_DEPTH, h_dim, sub), jnp.float32),
            pltpu.SemaphoreType.DMA((_DEPTH,)),
            pltpu.SemaphoreType.DMA((_DEPTH,)),
        ],
        cost_estimate=cost,
    )(w, b, gamma, beta, xt)
    return yt.T


def kernel(x, w, b, gamma, beta):
    return _encode(x, w, b, gamma, beta)


# speedup vs baseline: 1.5038x; 1.0253x over previous
import functools

import jax
import jax.numpy as jnp
from jax.experimental import pallas as pl
from jax.experimental.pallas import tpu as pltpu

_LN_EPS = 1e-5
_DEPTH = 6


def _ln_t_kernel(sub, n_chunks, w_ref, b_ref, g_ref, be_ref, x_ref, o_ref,
                 ibuf, obuf, isem, osem):
    h_dim = w_ref.shape[1]

    w_mat = w_ref[...]
    b_row = b_ref[...].reshape(1, h_dim)
    g_row = g_ref[...].reshape(1, h_dim)
    be_row = be_ref[...].reshape(1, h_dim)

    wcg = (w_mat - jnp.mean(w_mat, axis=1, keepdims=True)) * g_row
    bg_row = (b_row - jnp.mean(b_row)) * g_row
    g2 = jnp.maximum(g_row * g_row, jnp.float32(1e-30))
    gm_row = 1.0 / (h_dim * g2)
    bg_col = bg_row.reshape(h_dim, 1)
    be_col = be_row.reshape(h_dim, 1)

    def in_cp(gc):
        slot = jax.lax.rem(gc, _DEPTH)
        return pltpu.make_async_copy(
            x_ref.at[:, pl.ds(gc * sub, sub)], ibuf.at[slot], isem.at[slot])

    def out_cp(gc):
        slot = jax.lax.rem(gc, _DEPTH)
        return pltpu.make_async_copy(
            obuf.at[slot], o_ref.at[:, pl.ds(gc * sub, sub)], osem.at[slot])

    for k in range(_DEPTH):
        in_cp(k).start()

    def chunk(gc, _):
        slot = jax.lax.rem(gc, _DEPTH)
        in_cp(gc).wait()

        @pl.when(gc >= _DEPTH)
        def _():
            out_cp(gc - _DEPTH).wait()

        xs = ibuf[slot]
        d = jax.lax.dot_general(wcg, xs, (((0,), (0,)), ((), ())),
                                preferred_element_type=jnp.float32)
        d = d + bg_col
        var = jnp.dot(gm_row, d * d, preferred_element_type=jnp.float32)
        r = jax.lax.rsqrt(var + _LN_EPS)
        obuf[slot] = jnp.maximum(d * r + be_col, 0.0)
        out_cp(gc).start()

        @pl.when(gc + _DEPTH < n_chunks)
        def _():
            in_cp(gc + _DEPTH).start()
        return _

    jax.lax.fori_loop(0, n_chunks, chunk, None)

    for k in range(_DEPTH):
        out_cp(n_chunks - _DEPTH + k).wait()


@functools.partial(jax.jit, static_argnames=("sub_block",))
def _encode(x, w, b, gamma, beta, *, sub_block=32768):
    batch, a_dim = x.shape
    h_dim = w.shape[1]

    xt = x.T

    cost = pl.CostEstimate(
        flops=2 * batch * a_dim * h_dim,
        transcendentals=batch,
        bytes_accessed=4 * (batch * (a_dim + h_dim) + a_dim * h_dim + 3 * h_dim),
    )

    sub = min(sub_block, batch)
    while batch % sub:
        sub //= 2
    n_chunks = batch // sub
    body = functools.partial(_ln_t_kernel, sub, n_chunks)
    yt = pl.pallas_call(
        body,
        out_shape=jax.ShapeDtypeStruct((h_dim, batch), jnp.float32),
        in_specs=[
            pl.BlockSpec((a_dim, h_dim), lambda: (0, 0)),
            pl.BlockSpec((h_dim,), lambda: (0,)),
            pl.BlockSpec((h_dim,), lambda: (0,)),
            pl.BlockSpec((h_dim,), lambda: (0,)),
            pl.BlockSpec(memory_space=pltpu.MemorySpace.HBM),
        ],
        out_specs=pl.BlockSpec(memory_space=pltpu.MemorySpace.HBM),
        scratch_shapes=[
            pltpu.VMEM((_DEPTH, a_dim, sub), jnp.float32),
            pltpu.VMEM((_DEPTH, h_dim, sub), jnp.float32),
            pltpu.SemaphoreType.DMA((_DEPTH,)),
            pltpu.SemaphoreType.DMA((_DEPTH,)),
        ],
        cost_estimate=cost,
    )(w, b, gamma, beta, xt)
    return yt.T


def kernel(x, w, b, gamma, beta):
    return _encode(x, w, b, gamma, beta)
